# Initial kernel scaffold; baseline (speedup 1.0000x reference)
import jax
import jax.numpy as jnp
from jax.experimental import pallas as pl


def kernel(x, edge_index, batch, W1a, b1a, g1, be1, W1b, b1b, W2a, b2a, g2, be2, W2b, b2b, W3a, b3a, g3, be3, W3b, b3b, lin1_W, lin1_b, lin2_W, lin2_b):
    raise NotImplementedError("write your pallas kernel here")


# speedup vs baseline: 3.2112x; 3.2112x over previous
import functools

import jax
import jax.numpy as jnp
from jax import lax
from jax.experimental import pallas as pl
from jax.experimental.pallas import tpu as pltpu
from jax.experimental.pallas import tpu_sc as plsc

N = 10000
D = 128
E = 320000
G = 64
BN_EPS = 1e-5

NC = 2
NS = 16
NW = NC * NS
CHUNK = 128
CPW = 80
NPAIR = CPW // 2
EP = NW * CPW * CHUNK
NPAD = 10240
RPT = NPAD // NS

BR = 1000
NB = N // BR


def _sc_agg_body(h_hbm, src_hbm, dst_hbm, p_hbm,
                 agg_sh, src_v, dst_v, rows_v, sem0, sem1, dsem0, dsem1):
    c = lax.axis_index("c")
    s = lax.axis_index("s")
    wid = s * NC + c

    pltpu.sync_copy(src_hbm.at[wid], src_v)

    def _zrow(r, carry):
        for k in range(8):
            rows_v[0, r, pl.ds(k * 16, 16)] = jnp.zeros((16,), jnp.float32)
        return carry
    lax.fori_loop(0, CHUNK, _zrow, 0)
    zbase = s * RPT
    for j in range(RPT // CHUNK):
        pltpu.sync_copy(rows_v.at[0], agg_sh.at[pl.ds(zbase + j * CHUNK, CHUNK)])
    plsc.subcore_barrier()

    def _dst_cp(j, par):
        return pltpu.make_async_copy(dst_hbm.at[wid, j], dst_v.at[par], (dsem0, dsem1)[par])

    def _gather(j, par):
        return pltpu.make_async_copy(
            h_hbm.at[src_v.at[j]], rows_v.at[par], (sem0, sem1)[par])

    _dst_cp(0, 0).start()
    _dst_cp(1, 1).start()
    _gather(0, 0).start()

    def _pair(jj, carry):
        j0 = 2 * jj
        _gather(j0 + 1, 1).start()
        _gather(j0, 0).wait()
        _dst_cp(j0, 0).wait()
        pltpu.sync_copy(rows_v.at[0], agg_sh.at[dst_v.at[0]], add=True)

        @pl.when(jj < NPAIR - 1)
        def _():
            _dst_cp(j0 + 2, 0).start()
            _gather(j0 + 2, 0).start()

        _gather(j0 + 1, 1).wait()
        _dst_cp(j0 + 1, 1).wait()
        pltpu.sync_copy(rows_v.at[1], agg_sh.at[dst_v.at[1]], add=True)

        @pl.when(jj < NPAIR - 1)
        def _():
            _dst_cp(j0 + 3, 1).start()
        return carry

    lax.fori_loop(0, NPAIR, _pair, 0)
    plsc.subcore_barrier()

    rbase = s * RPT
    for j in range(RPT // CHUNK):
        off = rbase + j * CHUNK
        pltpu.sync_copy(agg_sh.at[pl.ds(off, CHUNK)], rows_v.at[0])
        pltpu.sync_copy(rows_v.at[0], p_hbm.at[c, pl.ds(off, CHUNK)])


@functools.cache
def _get_sc_agg():
    return functools.partial(
        pl.kernel,
        out_type=jax.ShapeDtypeStruct((NC, NPAD, D), jnp.float32),
        mesh=plsc.VectorSubcoreMesh(core_axis_name="c", subcore_axis_name="s",
                                    num_cores=NC, num_subcores=NS),
        scratch_types=[
            pltpu.VMEM_SHARED((NPAD, D), jnp.float32),
            pltpu.VMEM((CPW, CHUNK), jnp.int32),
            pltpu.VMEM((2, CHUNK), jnp.int32),
            pltpu.VMEM((2, CHUNK, D), jnp.float32),
            pltpu.SemaphoreType.DMA,
            pltpu.SemaphoreType.DMA,
            pltpu.SemaphoreType.DMA,
            pltpu.SemaphoreType.DMA,
        ],
    )(_sc_agg_body)


def _sc_agg(h, src3, dst3):
    return _get_sc_agg()(h, src3, dst3)


def _mlp_a_body(h_ref, p_ref, wa_ref, ba_ref, g_ref, be_ref,
                z_ref, st_ref, acc_ref):
    i = pl.program_id(0)
    hin = h_ref[...] + p_ref[0] + p_ref[1]
    z = jnp.dot(hin, wa_ref[...], preferred_element_type=jnp.float32)
    z = z + ba_ref[...]
    z_ref[...] = z
    ps = jnp.sum(z, axis=0, keepdims=True)
    ps2 = jnp.sum(z * z, axis=0, keepdims=True)

    @pl.when(i == 0)
    def _():
        acc_ref[...] = jnp.zeros_like(acc_ref)

    acc_ref[0:1, :] += ps
    acc_ref[1:2, :] += ps2

    @pl.when(i == NB - 1)
    def _():
        mu = acc_ref[0:1, :] / N
        var = acc_ref[1:2, :] / N - mu * mu
        scale = g_ref[...] * lax.rsqrt(var + BN_EPS)
        shift = be_ref[...] - mu * scale
        st_ref[...] = jnp.concatenate(
            [scale, shift, jnp.zeros((6, D), jnp.float32)], axis=0)


def _mlp_a(h, p, wa, ba, g, be):
    return pl.pallas_call(
        _mlp_a_body,
        grid=(NB,),
        in_specs=[
            pl.BlockSpec((BR, D), lambda i: (i, 0)),
            pl.BlockSpec((NC, BR, D), lambda i: (0, i, 0)),
            pl.BlockSpec((D, D), lambda i: (0, 0)),
            pl.BlockSpec((1, D), lambda i: (0, 0)),
            pl.BlockSpec((1, D), lambda i: (0, 0)),
            pl.BlockSpec((1, D), lambda i: (0, 0)),
        ],
        out_specs=[
            pl.BlockSpec((BR, D), lambda i: (i, 0)),
            pl.BlockSpec((8, D), lambda i: (0, 0)),
        ],
        out_shape=[
            jax.ShapeDtypeStruct((N, D), jnp.float32),
            jax.ShapeDtypeStruct((8, D), jnp.float32),
        ],
        scratch_shapes=[pltpu.VMEM((8, D), jnp.float32)],
    )(h, p, wa, ba, g, be)


def _mlp_b_body(z_ref, st_ref, wb_ref, bb_ref, h_ref):
    a = jax.nn.relu(z_ref[...] * st_ref[0:1, :] + st_ref[1:2, :])
    h = jnp.dot(a, wb_ref[...], preferred_element_type=jnp.float32)
    h_ref[...] = jax.nn.relu(h + bb_ref[...])


def _mlp_b(z, st, wb, bb):
    return pl.pallas_call(
        _mlp_b_body,
        grid=(NB,),
        in_specs=[
            pl.BlockSpec((BR, D), lambda i: (i, 0)),
            pl.BlockSpec((8, D), lambda i: (0, 0)),
            pl.BlockSpec((D, D), lambda i: (0, 0)),
            pl.BlockSpec((1, D), lambda i: (0, 0)),
        ],
        out_specs=pl.BlockSpec((BR, D), lambda i: (i, 0)),
        out_shape=jax.ShapeDtypeStruct((N, D), jnp.float32),
    )(z, st, wb, bb)


def _mlp_b3_body(z_ref, st_ref, wb_ref, bb_ref, batch_ref,
                 l1w_ref, l1b_ref, l2w_ref, l2b_ref, y_ref, acc_ref):
    i = pl.program_id(0)
    a = jax.nn.relu(z_ref[...] * st_ref[0:1, :] + st_ref[1:2, :])
    h = jnp.dot(a, wb_ref[...], preferred_element_type=jnp.float32)
    h = jax.nn.relu(h + bb_ref[...])
    seg = batch_ref[0]
    onehot = (lax.broadcasted_iota(jnp.int32, (G, BR), 0) == seg
              ).astype(jnp.float32)
    pooled = jnp.dot(onehot, h, preferred_element_type=jnp.float32)

    @pl.when(i == 0)
    def _():
        acc_ref[...] = jnp.zeros_like(acc_ref)

    acc_ref[...] += pooled

    @pl.when(i == NB - 1)
    def _():
        t = jax.nn.relu(
            jnp.dot(acc_ref[...], l1w_ref[...],
                    preferred_element_type=jnp.float32) + l1b_ref[...])
        y_ref[...] = jnp.dot(t, l2w_ref[...],
                             preferred_element_type=jnp.float32) + l2b_ref[...]


def _mlp_b3(z, st, wb, bb, batch3, l1w, l1b, l2w, l2b):
    return pl.pallas_call(
        _mlp_b3_body,
        grid=(NB,),
        in_specs=[
            pl.BlockSpec((BR, D), lambda i: (i, 0)),
            pl.BlockSpec((8, D), lambda i: (0, 0)),
            pl.BlockSpec((D, D), lambda i: (0, 0)),
            pl.BlockSpec((1, D), lambda i: (0, 0)),
            pl.BlockSpec((1, 1, BR), lambda i: (i, 0, 0)),
            pl.BlockSpec((D, D), lambda i: (0, 0)),
            pl.BlockSpec((1, D), lambda i: (0, 0)),
            pl.BlockSpec((D, D), lambda i: (0, 0)),
            pl.BlockSpec((1, D), lambda i: (0, 0)),
        ],
        out_specs=pl.BlockSpec((G, D), lambda i: (0, 0)),
        out_shape=jax.ShapeDtypeStruct((G, D), jnp.float32),
        scratch_shapes=[pltpu.VMEM((G, D), jnp.float32)],
    )(z, st, wb, bb, batch3, l1w, l1b, l2w, l2b)


def kernel(x, edge_index, batch,
           W1a, b1a, g1, be1, W1b, b1b,
           W2a, b2a, g2, be2, W2b, b2b,
           W3a, b3a, g3, be3, W3b, b3b,
           lin1_W, lin1_b, lin2_W, lin2_b):
    src = edge_index[0]
    dst = edge_index[1]
    pad = EP - E
    srcp = jnp.concatenate([src, jnp.zeros((pad,), src.dtype)])
    dstp = jnp.concatenate([dst, jnp.full((pad,), N, dst.dtype)])
    src3 = srcp.reshape(NW, CPW, CHUNK).astype(jnp.int32)
    dst3 = dstp.reshape(NW, CPW, CHUNK).astype(jnp.int32)

    batch3 = batch.astype(jnp.int32).reshape(NB, 1, BR)
    r1 = lambda v: v.reshape(1, D)
    l2w = jnp.pad(lin2_W, ((0, 0), (0, D - 1)))
    l2b = jnp.pad(lin2_b, (0, D - 1)).reshape(1, D)

    h = x
    p = _sc_agg(h, src3, dst3)
    z, st = _mlp_a(h, p, W1a, r1(b1a), r1(g1), r1(be1))
    h = _mlp_b(z, st, W1b, r1(b1b))

    p = _sc_agg(h, src3, dst3)
    z, st = _mlp_a(h, p, W2a, r1(b2a), r1(g2), r1(be2))
    h = _mlp_b(z, st, W2b, r1(b2b))

    p = _sc_agg(h, src3, dst3)
    z, st = _mlp_a(h, p, W3a, r1(b3a), r1(g3), r1(be3))
    y = _mlp_b3(z, st, W3b, r1(b3b), batch3, lin1_W, r1(lin1_b), l2w, l2b)

    return y[:, :1]


# speedup vs baseline: 12.7233x; 3.9622x over previous
import functools

import jax
import jax.numpy as jnp
from jax import lax
from jax.experimental import pallas as pl
from jax.experimental.pallas import tpu as pltpu
from jax.experimental.pallas import tpu_sc as plsc

N = 10000
D = 128
E = 320000
G = 64
BN_EPS = 1e-5

NC = 2
NS = 16
NW = NC * NS
CHUNK = 128
CPW = 80
NPAIR = CPW // 2
EP = NW * CPW * CHUNK
NPAD = 10240
RPT = NPAD // NS

BR = 1000
NB = N // BR


def _sc_agg_body(h_hbm, src_hbm, dst_hbm, p_hbm,
                 agg_sh, src_v, dst_v, rows_v, sem0, sem1, dsem0, dsem1):
    c = lax.axis_index("c")
    s = lax.axis_index("s")
    wid = s * NC + c

    idx_cp = pltpu.make_async_copy(src_hbm.at[wid], src_v, dsem0)
    idx_cp.start()

    def _zrow(r, carry):
        for k in range(8):
            rows_v[0, r, pl.ds(k * 16, 16)] = jnp.zeros((16,), jnp.float32)
        return carry
    lax.fori_loop(0, CHUNK, _zrow, 0)
    zbase = s * RPT
    zcps = [
        pltpu.make_async_copy(
            rows_v.at[0], agg_sh.at[pl.ds(zbase + j * CHUNK, CHUNK)], sem1)
        for j in range(RPT // CHUNK)
    ]
    for cp in zcps:
        cp.start()
    idx_cp.wait()
    for cp in zcps:
        cp.wait()
    plsc.subcore_barrier()

    def _dst_cp(j, par):
        return pltpu.make_async_copy(dst_hbm.at[wid, j], dst_v.at[par], (dsem0, dsem1)[par])

    def _gather(j, par):
        return pltpu.make_async_copy(
            h_hbm.at[src_v.at[j]], rows_v.at[par], (sem0, sem1)[par])

    _dst_cp(0, 0).start()
    _dst_cp(1, 1).start()
    _gather(0, 0).start()

    def _pair(jj, carry):
        j0 = 2 * jj
        _gather(j0 + 1, 1).start()
        _gather(j0, 0).wait()
        _dst_cp(j0, 0).wait()
        pltpu.sync_copy(rows_v.at[0], agg_sh.at[dst_v.at[0]], add=True)

        @pl.when(jj < NPAIR - 1)
        def _():
            _dst_cp(j0 + 2, 0).start()
            _gather(j0 + 2, 0).start()

        _gather(j0 + 1, 1).wait()
        _dst_cp(j0 + 1, 1).wait()
        pltpu.sync_copy(rows_v.at[1], agg_sh.at[dst_v.at[1]], add=True)

        @pl.when(jj < NPAIR - 1)
        def _():
            _dst_cp(j0 + 3, 1).start()
        return carry

    lax.fori_loop(0, NPAIR, _pair, 0)
    plsc.subcore_barrier()

    rbase = s * RPT
    rcps = [
        pltpu.make_async_copy(
            agg_sh.at[pl.ds(rbase + j * CHUNK, CHUNK)],
            p_hbm.at[c, pl.ds(rbase + j * CHUNK, CHUNK)], sem0)
        for j in range(RPT // CHUNK)
    ]
    for cp in rcps:
        cp.start()
    for cp in rcps:
        cp.wait()


@functools.cache
def _get_sc_agg():
    return functools.partial(
        pl.kernel,
        out_type=jax.ShapeDtypeStruct((NC, NPAD, D), jnp.float32),
        mesh=plsc.VectorSubcoreMesh(core_axis_name="c", subcore_axis_name="s",
                                    num_cores=NC, num_subcores=NS),
        scratch_types=[
            pltpu.VMEM_SHARED((NPAD, D), jnp.float32),
            pltpu.VMEM((CPW, CHUNK), jnp.int32),
            pltpu.VMEM((2, CHUNK), jnp.int32),
            pltpu.VMEM((2, CHUNK, D), jnp.float32),
            pltpu.SemaphoreType.DMA,
            pltpu.SemaphoreType.DMA,
            pltpu.SemaphoreType.DMA,
            pltpu.SemaphoreType.DMA,
        ],
    )(_sc_agg_body)


def _sc_agg(h, src3, dst3):
    return _get_sc_agg()(h, src3, dst3)


def _mlp_phase0(h_ref, p_ref, wa_ref, ba_ref, g_ref, be_ref,
                z_scr, acc_ref, st_ref, i):
    hin = h_ref[...] + p_ref[0] + p_ref[1]
    z = jnp.dot(hin, wa_ref[...], preferred_element_type=jnp.float32)
    z = z + ba_ref[...]
    z_scr[pl.ds(i * BR, BR), :] = z

    @pl.when(i == 0)
    def _():
        acc_ref[...] = jnp.zeros_like(acc_ref)

    acc_ref[0:1, :] += jnp.sum(z, axis=0, keepdims=True)
    acc_ref[1:2, :] += jnp.sum(z * z, axis=0, keepdims=True)

    @pl.when(i == NB - 1)
    def _():
        mu = acc_ref[0:1, :] / N
        var = acc_ref[1:2, :] / N - mu * mu
        scale = g_ref[...] * lax.rsqrt(var + BN_EPS)
        shift = be_ref[...] - mu * scale
        st_ref[0:1, :] = scale
        st_ref[1:2, :] = shift


def _mlp_second(z_scr, st_ref, wb_ref, bb_ref, i):
    z = z_scr[pl.ds(i * BR, BR), :]
    a = jax.nn.relu(z * st_ref[0:1, :] + st_ref[1:2, :])
    h = jnp.dot(a, wb_ref[...], preferred_element_type=jnp.float32)
    return jax.nn.relu(h + bb_ref[...])


def _mlp_body(h_ref, p_ref, wa_ref, ba_ref, g_ref, be_ref, wb_ref, bb_ref,
              out_ref, z_scr, acc_ref, st_ref):
    ph = pl.program_id(0)
    i = pl.program_id(1)

    @pl.when(ph == 0)
    def _():
        _mlp_phase0(h_ref, p_ref, wa_ref, ba_ref, g_ref, be_ref,
                    z_scr, acc_ref, st_ref, i)

    @pl.when(ph == 1)
    def _():
        out_ref[...] = _mlp_second(z_scr, st_ref, wb_ref, bb_ref, i)


def _mlp(h, p, wa, ba, g, be, wb, bb):
    full = lambda shape: pl.BlockSpec(shape, lambda ph, i: (0,) * len(shape))
    return pl.pallas_call(
        _mlp_body,
        grid=(2, NB),
        in_specs=[
            pl.BlockSpec((BR, D), lambda ph, i: (i * (1 - ph), 0)),
            pl.BlockSpec((NC, BR, D), lambda ph, i: (0, i * (1 - ph), 0)),
            full((D, D)), full((1, D)), full((1, D)), full((1, D)),
            full((D, D)), full((1, D)),
        ],
        out_specs=pl.BlockSpec((BR, D), lambda ph, i: (i, 0)),
        out_shape=jax.ShapeDtypeStruct((N, D), jnp.float32),
        scratch_shapes=[
            pltpu.VMEM((N, D), jnp.float32),
            pltpu.VMEM((8, D), jnp.float32),
            pltpu.VMEM((8, D), jnp.float32),
        ],
    )(h, p, wa, ba, g, be, wb, bb)


def _mlp3_body(h_ref, p_ref, wa_ref, ba_ref, g_ref, be_ref, wb_ref, bb_ref,
               batch_ref, l1w_ref, l1b_ref, l2w_ref, l2b_ref,
               y_ref, z_scr, acc_ref, st_ref, pool_ref):
    ph = pl.program_id(0)
    i = pl.program_id(1)

    @pl.when(ph == 0)
    def _():
        _mlp_phase0(h_ref, p_ref, wa_ref, ba_ref, g_ref, be_ref,
                    z_scr, acc_ref, st_ref, i)

    @pl.when(ph == 1)
    def _():
        h = _mlp_second(z_scr, st_ref, wb_ref, bb_ref, i)
        seg = batch_ref[0]
        onehot = (lax.broadcasted_iota(jnp.int32, (G, BR), 0) == seg
                  ).astype(jnp.float32)
        pooled = jnp.dot(onehot, h, preferred_element_type=jnp.float32)

        @pl.when(i == 0)
        def _():
            pool_ref[...] = jnp.zeros_like(pool_ref)

        pool_ref[...] += pooled

        @pl.when(i == NB - 1)
        def _():
            t = jax.nn.relu(
                jnp.dot(pool_ref[...], l1w_ref[...],
                        preferred_element_type=jnp.float32) + l1b_ref[...])
            y_ref[...] = jnp.dot(t, l2w_ref[...],
                                 preferred_element_type=jnp.float32) + l2b_ref[...]


def _mlp3(h, p, wa, ba, g, be, wb, bb, batch3, l1w, l1b, l2w, l2b):
    full = lambda shape: pl.BlockSpec(shape, lambda ph, i: (0,) * len(shape))
    return pl.pallas_call(
        _mlp3_body,
        grid=(2, NB),
        in_specs=[
            pl.BlockSpec((BR, D), lambda ph, i: (i * (1 - ph), 0)),
            pl.BlockSpec((NC, BR, D), lambda ph, i: (0, i * (1 - ph), 0)),
            full((D, D)), full((1, D)), full((1, D)), full((1, D)),
            full((D, D)), full((1, D)),
            pl.BlockSpec((1, 1, BR), lambda ph, i: (i, 0, 0)),
            full((D, D)), full((1, D)), full((D, D)), full((1, D)),
        ],
        out_specs=pl.BlockSpec((G, D), lambda ph, i: (0, 0)),
        out_shape=jax.ShapeDtypeStruct((G, D), jnp.float32),
        scratch_shapes=[
            pltpu.VMEM((N, D), jnp.float32),
            pltpu.VMEM((8, D), jnp.float32),
            pltpu.VMEM((8, D), jnp.float32),
            pltpu.VMEM((G, D), jnp.float32),
        ],
    )(h, p, wa, ba, g, be, wb, bb, batch3, l1w, l1b, l2w, l2b)


def kernel(x, edge_index, batch,
           W1a, b1a, g1, be1, W1b, b1b,
           W2a, b2a, g2, be2, W2b, b2b,
           W3a, b3a, g3, be3, W3b, b3b,
           lin1_W, lin1_b, lin2_W, lin2_b):
    src = edge_index[0]
    dst = edge_index[1]
    pad = EP - E
    pad_src = (jnp.arange(pad, dtype=jnp.int32) * 97) % N
    pad_dst = N + (jnp.arange(pad, dtype=jnp.int32) % (NPAD - N))
    srcp = jnp.concatenate([src, pad_src.astype(src.dtype)])
    dstp = jnp.concatenate([dst, pad_dst.astype(dst.dtype)])
    src3 = srcp.reshape(NW, CPW, CHUNK).astype(jnp.int32)
    dst3 = dstp.reshape(NW, CPW, CHUNK).astype(jnp.int32)

    batch3 = batch.astype(jnp.int32).reshape(NB, 1, BR)
    r1 = lambda v: v.reshape(1, D)
    l2w = jnp.pad(lin2_W, ((0, 0), (0, D - 1)))
    l2b = jnp.pad(lin2_b, (0, D - 1)).reshape(1, D)

    h = x
    p = _sc_agg(h, src3, dst3)
    h = _mlp(h, p, W1a, r1(b1a), r1(g1), r1(be1), W1b, r1(b1b))

    p = _sc_agg(h, src3, dst3)
    h = _mlp(h, p, W2a, r1(b2a), r1(g2), r1(be2), W2b, r1(b2b))

    p = _sc_agg(h, src3, dst3)
    y = _mlp3(h, p, W3a, r1(b3a), r1(g3), r1(be3), W3b, r1(b3b),
              batch3, lin1_W, r1(lin1_b), l2w, l2b)

    return y[:, :1]
